# Initial kernel scaffold; baseline (speedup 1.0000x reference)
import jax
import jax.numpy as jnp
from jax.experimental import pallas as pl


def kernel(x, edge_index, batch, W1, b1, W2, b2, W3, b3, W4, b4, Wr1, Wn1, bp1, Wr2, Wn2, bp2, Wr3, Wn3, bp3, Wr4, Wn4, bp4):
    raise NotImplementedError("write your pallas kernel here")


# speedup vs baseline: 16.7297x; 16.7297x over previous
import functools
import math

import jax
import jax.numpy as jnp
from jax import lax
from jax.experimental import pallas as pl
from jax.experimental.pallas import tpu as pltpu
from jax.experimental.pallas import tpu_sc as plsc

N = 10000
E = 320000
H = 64
HP = 128
NPAD = 10240
NTILES = 32
CHUNK = 128
CHUNKS_PER_TILE = 79
EPT = CHUNK * CHUNKS_PER_TILE
EP = EPT * NTILES
ROWS_PER_TILE = NPAD // 16
K_SIZES = (5000, 2500, 1250, 625)


@functools.cache
def _get_mesh():
    return plsc.VectorSubcoreMesh(core_axis_name="c", subcore_axis_name="s",
                                  num_cores=2, num_subcores=16)


def _sweep_rows_body(src_hbm, dst_hbm, tab_hbm, out_hbm, acc, srcv, dstv, rows):
    cid = lax.axis_index("c")
    sid = lax.axis_index("s")
    wid = cid * 16 + sid

    @pl.loop(0, CHUNK)
    def _(i):
        for j in range(HP // 16):
            rows[i, pl.ds(j * 16, 16)] = jnp.zeros((16,), jnp.float32)

    @pl.loop(0, ROWS_PER_TILE // CHUNK)
    def _(i):
        pltpu.sync_copy(rows, acc.at[pl.ds(sid * ROWS_PER_TILE + i * CHUNK, CHUNK)])

    plsc.subcore_barrier()

    @pl.loop(0, CHUNKS_PER_TILE)
    def _(k):
        base = wid * EPT + k * CHUNK
        pltpu.sync_copy(src_hbm.at[pl.ds(base, CHUNK)], srcv)
        pltpu.sync_copy(dst_hbm.at[pl.ds(base, CHUNK)], dstv)
        pltpu.sync_copy(tab_hbm.at[srcv], rows)
        pltpu.sync_copy(rows, acc.at[dstv], add=True)

    plsc.subcore_barrier()
    pltpu.sync_copy(acc.at[pl.ds(sid * ROWS_PER_TILE, ROWS_PER_TILE)],
                    out_hbm.at[cid, pl.ds(sid * ROWS_PER_TILE, ROWS_PER_TILE)])


def _sweep_rows(srcp, dstp, tab):
    return pl.kernel(
        _sweep_rows_body,
        out_type=jax.ShapeDtypeStruct((2, NPAD, HP), jnp.float32),
        mesh=_get_mesh(),
        scratch_types=[
            pltpu.VMEM_SHARED((NPAD, HP), jnp.float32),
            pltpu.VMEM((CHUNK,), jnp.int32),
            pltpu.VMEM((CHUNK,), jnp.int32),
            pltpu.VMEM((CHUNK, HP), jnp.float32),
        ],
    )(srcp, dstp, tab)


def _sweep_scalar_body(src_hbm, dst_hbm, tab_hbm, out_hbm,
                       acc, keptt, srcv, dstv, vals):
    cid = lax.axis_index("c")
    sid = lax.axis_index("s")
    wid = cid * 16 + sid

    pltpu.sync_copy(tab_hbm, keptt)

    @pl.loop(0, CHUNK, step=16)
    def _(i):
        vals[pl.ds(i, 16)] = jnp.zeros((16,), jnp.float32)

    @pl.loop(0, ROWS_PER_TILE // CHUNK)
    def _(i):
        pltpu.sync_copy(vals, acc.at[pl.ds(sid * ROWS_PER_TILE + i * CHUNK, CHUNK)])

    plsc.subcore_barrier()

    @pl.loop(0, CHUNKS_PER_TILE)
    def _(k):
        base = wid * EPT + k * CHUNK
        pltpu.sync_copy(src_hbm.at[pl.ds(base, CHUNK)], srcv)
        pltpu.sync_copy(dst_hbm.at[pl.ds(base, CHUNK)], dstv)

        @pl.loop(0, CHUNK, step=16)
        def _(j):
            s16 = srcv[pl.ds(j, 16)]
            vals[pl.ds(j, 16)] = plsc.load_gather(keptt, [s16])

        pltpu.sync_copy(vals, acc.at[dstv], add=True)

    plsc.subcore_barrier()
    pltpu.sync_copy(acc.at[pl.ds(sid * ROWS_PER_TILE, ROWS_PER_TILE)],
                    out_hbm.at[cid, pl.ds(sid * ROWS_PER_TILE, ROWS_PER_TILE)])


def _sweep_scalar(srcp, dstp, tab):
    return pl.kernel(
        _sweep_scalar_body,
        out_type=jax.ShapeDtypeStruct((2, NPAD), jnp.float32),
        mesh=_get_mesh(),
        scratch_types=[
            pltpu.VMEM_SHARED((NPAD,), jnp.float32),
            pltpu.VMEM((NPAD,), jnp.float32),
            pltpu.VMEM((CHUNK,), jnp.int32),
            pltpu.VMEM((CHUNK,), jnp.int32),
            pltpu.VMEM((CHUNK,), jnp.float32),
        ],
        compiler_params=pltpu.CompilerParams(needs_layout_passes=False),
    )(srcp, dstp, tab)


def _mm_body(x_ref, w_ref, o_ref):
    o_ref[...] = jnp.dot(x_ref[...], w_ref[...],
                         preferred_element_type=jnp.float32)


def _tc_mm(x, w):
    return pl.pallas_call(
        _mm_body,
        out_shape=jax.ShapeDtypeStruct((x.shape[0], w.shape[1]), jnp.float32),
    )(x, w)


def _scale_body(xw_ref, kept_ref, degs_ref, y_ref, self_ref, g_ref):
    xw = xw_ref[...]
    kept = kept_ref[...]
    deg = 1.0 + kept * (degs_ref[0] + degs_ref[1])
    g = kept * lax.rsqrt(deg)
    y_ref[...] = g * xw
    self_ref[...] = xw / deg
    g_ref[...] = g


def _tc_scale(xw, kept, degs):
    return pl.pallas_call(
        _scale_body,
        out_shape=[
            jax.ShapeDtypeStruct((NPAD, HP), jnp.float32),
            jax.ShapeDtypeStruct((NPAD, HP), jnp.float32),
            jax.ShapeDtypeStruct((NPAD, 1), jnp.float32),
        ],
    )(xw, kept, degs)


def _post_body(s_ref, self_ref, g_ref, b_ref, kept_ref, xn_ref, y2_ref):
    xn = jnp.maximum(
        g_ref[...] * (s_ref[0] + s_ref[1]) + self_ref[...] + b_ref[...], 0.0)
    xn_ref[...] = xn
    y2_ref[...] = xn * kept_ref[...]


def _tc_post(s, selfterm, g, b, kept):
    return pl.pallas_call(
        _post_body,
        out_shape=[
            jax.ShapeDtypeStruct((NPAD, HP), jnp.float32),
            jax.ShapeDtypeStruct((NPAD, HP), jnp.float32),
        ],
    )(s, selfterm, g, b, kept)


def _score_body(xn_ref, t_ref, kept_ref, wr_ref, wn_ref, bp_ref, score_ref):
    neigh = kept_ref[...] * (t_ref[0] + t_ref[1])
    s = (jnp.sum(xn_ref[...] * wr_ref[...], axis=1, keepdims=True)
         + jnp.sum(neigh * wn_ref[...], axis=1, keepdims=True) + bp_ref[...])
    score_ref[...] = s


def _tc_score(xn, t, kept, wr, wn, bp):
    return pl.pallas_call(
        _score_body,
        out_shape=jax.ShapeDtypeStruct((NPAD, 1), jnp.float32),
    )(xn, t, kept, wr, wn, bp.reshape(1, 1))


def _sortable(f):
    bits = lax.bitcast_convert_type(f, jnp.int32)
    return bits ^ (lax.shift_right_arithmetic(bits, 31) & jnp.int32(0x7FFFFFFF))


def _topk_body(k, sg_ref, kg_ref, out_ref):
    imin = jnp.int32(-2147483648)
    key = jnp.where(kg_ref[...] > 0, _sortable(sg_ref[...]), imin)

    def sbody(bi, t_biased):
        cand = t_biased | lax.shift_left(jnp.int32(1), 31 - bi)
        t_signed = cand ^ imin
        cnt = jnp.sum((key >= t_signed).astype(jnp.int32))
        return jnp.where(cnt >= k, cand, t_biased)

    tb = lax.fori_loop(0, 32, sbody, jnp.int32(0))
    t = tb ^ imin
    strict = jnp.sum((key > t).astype(jnp.int32))
    r = k - strict
    tie = key == t
    idx = (lax.broadcasted_iota(jnp.int32, (80, 128), 0) * 128
           + lax.broadcasted_iota(jnp.int32, (80, 128), 1))
    u = jnp.where(tie, jnp.int32(16383) - idx, jnp.int32(-1))

    def ubody(bi, tu):
        cand = tu | lax.shift_left(jnp.int32(1), 13 - bi)
        cnt = jnp.sum((u >= cand).astype(jnp.int32))
        return jnp.where(cnt >= r, cand, tu)

    tu = lax.fori_loop(0, 14, ubody, jnp.int32(0))
    tau = jnp.where(r > 0, jnp.int32(16383) - tu, jnp.int32(-1))
    out_ref[0, 0] = t
    out_ref[0, 1] = tau
    out_ref[0, 2] = r


def _tc_topk(sg, kg, k):
    return pl.pallas_call(
        functools.partial(_topk_body, k),
        out_shape=jax.ShapeDtypeStruct((1, 4), jnp.int32),
        out_specs=pl.BlockSpec(memory_space=pltpu.SMEM),
    )(sg, kg)


def _gate_body(k, xn_ref, score_ref, kept_ref, scal_ref, xf_ref, kn_ref, ro_ref):
    score = score_ref[...]
    key = jnp.where(kept_ref[...] > 0, _sortable(score), jnp.int32(-2147483648))
    t = scal_ref[0, 0]
    tau = scal_ref[0, 1]
    r = scal_ref[0, 2]
    idx = lax.broadcasted_iota(jnp.int32, (NPAD, 1), 0)
    keptn = ((key > t) | ((key == t) & (idx <= tau) & (r > 0))).astype(
        jnp.float32)
    xf = xn_ref[...] * jnp.tanh(score) * keptn
    xf_ref[...] = xf
    kn_ref[...] = keptn
    mean = jnp.sum(xf, axis=0, keepdims=True) * (1.0 / k)
    mx = jnp.max(jnp.where(keptn > 0, xf, -jnp.inf), axis=0, keepdims=True)
    ro_ref[...] = jnp.concatenate([mean[:, :H], mx[:, :H]], axis=1)


def _tc_gate(xn, score, kept, scal, k):
    return pl.pallas_call(
        functools.partial(_gate_body, k),
        out_shape=[
            jax.ShapeDtypeStruct((NPAD, HP), jnp.float32),
            jax.ShapeDtypeStruct((NPAD, 1), jnp.float32),
            jax.ShapeDtypeStruct((1, 2 * H), jnp.float32),
        ],
        in_specs=[
            pl.BlockSpec(memory_space=pltpu.VMEM),
            pl.BlockSpec(memory_space=pltpu.VMEM),
            pl.BlockSpec(memory_space=pltpu.VMEM),
            pl.BlockSpec(memory_space=pltpu.SMEM),
        ],
    )(xn, score, kept, scal)


def _pad_cols(w, rows=None):
    out_rows = w.shape[0] if rows is None else rows
    z = jnp.zeros((out_rows, HP), jnp.float32)
    return z.at[:w.shape[0], :w.shape[1]].set(w)


def kernel(x, edge_index, batch, W1, b1, W2, b2, W3, b3, W4, b4,
           Wr1, Wn1, bp1, Wr2, Wn2, bp2, Wr3, Wn3, bp3, Wr4, Wn4, bp4):
    src = edge_index[0]
    dst = edge_index[1]
    npad_e = EP - E
    pad_iota = jnp.arange(npad_e, dtype=jnp.int32) % 128
    srcp = jnp.concatenate([src, pad_iota])
    dstp = jnp.concatenate([dst, N + pad_iota])

    xp = jnp.zeros((NPAD, HP), jnp.float32).at[:N, :x.shape[1]].set(x)
    kept = jnp.zeros((NPAD, 1), jnp.float32).at[:N].set(1.0)

    Ws = ((_pad_cols(W1, 128), _pad_cols(b1.reshape(1, H))),
          (_pad_cols(W2, 128), _pad_cols(b2.reshape(1, H))),
          (_pad_cols(W3, 128), _pad_cols(b3.reshape(1, H))),
          (_pad_cols(W4, 128), _pad_cols(b4.reshape(1, H))))
    Ps = ((_pad_cols(Wr1.reshape(1, H)), _pad_cols(Wn1.reshape(1, H)), bp1),
          (_pad_cols(Wr2.reshape(1, H)), _pad_cols(Wn2.reshape(1, H)), bp2),
          (_pad_cols(Wr3.reshape(1, H)), _pad_cols(Wn3.reshape(1, H)), bp3),
          (_pad_cols(Wr4.reshape(1, H)), _pad_cols(Wn4.reshape(1, H)), bp4))

    out = jnp.zeros((1, 2 * H), jnp.float32)
    xc = xp
    for i in range(4):
        W, b = Ws[i]
        Wr, Wn, bp = Ps[i]
        k = K_SIZES[i]
        xw = _tc_mm(xc, W)
        degs = _sweep_scalar(srcp, dstp, kept.reshape(NPAD))
        y, selfterm, g = _tc_scale(xw, kept, degs.reshape(2, NPAD, 1))
        s = _sweep_rows(srcp, dstp, y)
        xn, y2 = _tc_post(s, selfterm, g, b, kept)
        t = _sweep_rows(srcp, dstp, y2)
        score = _tc_score(xn, t, kept, Wr, Wn, bp)
        scal = _tc_topk(score.reshape(80, 128), kept.reshape(80, 128), k)
        xc, kept, ro = _tc_gate(xn, score, kept, scal, k)
        out = out + ro
    return out


# speedup vs baseline: 30.9149x; 1.8479x over previous
import functools
import math

import jax
import jax.numpy as jnp
from jax import lax
from jax.experimental import pallas as pl
from jax.experimental.pallas import tpu as pltpu
from jax.experimental.pallas import tpu_sc as plsc

N = 10000
E = 320000
H = 64
HP = 128
NPAD = 10240
NTILES = 32
CHUNK = 128
CHUNKS_PER_TILE = 80
EPT = CHUNK * CHUNKS_PER_TILE
EP = EPT * NTILES
ROWS_PER_TILE = NPAD // 16
K_SIZES = (5000, 2500, 1250, 625)


@functools.cache
def _get_mesh():
    return plsc.VectorSubcoreMesh(core_axis_name="c", subcore_axis_name="s",
                                  num_cores=2, num_subcores=16)


def _sweep_rows_body(src_hbm, dst_hbm, tab_hbm, out_hbm, acc,
                     dstv, sa, sb, rows0, rows1,
                     sg0, sg1, ss0, ss1, sia, sib):
    cid = lax.axis_index("c")
    sid = lax.axis_index("s")
    wid = cid * 16 + sid
    CPT = CHUNKS_PER_TILE

    pltpu.sync_copy(dst_hbm.at[wid], dstv)

    @pl.loop(0, CHUNK)
    def _(i):
        for j in range(HP // 16):
            rows0[i, pl.ds(j * 16, 16)] = jnp.zeros((16,), jnp.float32)

    @pl.loop(0, ROWS_PER_TILE // CHUNK)
    def _(i):
        pltpu.sync_copy(rows0, acc.at[pl.ds(sid * ROWS_PER_TILE + i * CHUNK, CHUNK)])

    plsc.subcore_barrier()

    def idx_load(k, buf, sem):
        pltpu.async_copy(src_hbm.at[pl.ds(wid * EPT + k * CHUNK, CHUNK)],
                         buf, sem)

    def idx_wait(k, buf, sem):
        pltpu.make_async_copy(src_hbm.at[pl.ds(wid * EPT + k * CHUNK, CHUNK)],
                              buf, sem).wait()

    pltpu.sync_copy(src_hbm.at[pl.ds(wid * EPT, CHUNK)], sa)
    pltpu.async_copy(tab_hbm.at[sa], rows0, sg0)
    idx_load(1, sb, sib)

    @pl.loop(0, CPT, step=2)
    def _(k):
        pltpu.make_async_copy(tab_hbm.at[sa], rows0, sg0).wait()
        idx_wait(k + 1, sb, sib)

        @pl.when(k > 0)
        def _():
            pltpu.make_async_copy(rows1, acc.at[dstv.at[k - 1]], ss1).wait()

        pltpu.async_copy(tab_hbm.at[sb], rows1, sg1)
        pltpu.async_copy(rows0, acc.at[dstv.at[k]], ss0, add=True)

        @pl.when(k + 2 < CPT)
        def _():
            idx_load(k + 2, sa, sia)

        pltpu.make_async_copy(tab_hbm.at[sb], rows1, sg1).wait()
        pltpu.make_async_copy(rows0, acc.at[dstv.at[k]], ss0).wait()

        @pl.when(k + 2 < CPT)
        def _():
            idx_wait(k + 2, sa, sia)
            pltpu.async_copy(tab_hbm.at[sa], rows0, sg0)

        pltpu.async_copy(rows1, acc.at[dstv.at[k + 1]], ss1, add=True)

        @pl.when(k + 3 < CPT)
        def _():
            idx_load(k + 3, sb, sib)

    pltpu.make_async_copy(rows1, acc.at[dstv.at[CPT - 1]], ss1).wait()
    plsc.subcore_barrier()
    pltpu.sync_copy(acc.at[pl.ds(sid * ROWS_PER_TILE, ROWS_PER_TILE)],
                    out_hbm.at[cid, pl.ds(sid * ROWS_PER_TILE, ROWS_PER_TILE)])


def _sweep_rows(srcp, dstp, tab):
    return pl.kernel(
        _sweep_rows_body,
        out_type=jax.ShapeDtypeStruct((2, NPAD, HP), jnp.float32),
        mesh=_get_mesh(),
        scratch_types=[
            pltpu.VMEM_SHARED((NPAD, HP), jnp.float32),
            pltpu.VMEM((CHUNKS_PER_TILE, CHUNK), jnp.int32),
            pltpu.VMEM((CHUNK,), jnp.int32),
            pltpu.VMEM((CHUNK,), jnp.int32),
            pltpu.VMEM((CHUNK, HP), jnp.float32),
            pltpu.VMEM((CHUNK, HP), jnp.float32),
            pltpu.SemaphoreType.DMA,
            pltpu.SemaphoreType.DMA,
            pltpu.SemaphoreType.DMA,
            pltpu.SemaphoreType.DMA,
            pltpu.SemaphoreType.DMA,
            pltpu.SemaphoreType.DMA,
        ],
    )(srcp, dstp, tab)


def _sweep_scalar_body(src_hbm, dst_hbm, tab_hbm, out_hbm,
                       acc, keptt, srcv, dstv, vals0, vals1, ss0, ss1):
    cid = lax.axis_index("c")
    sid = lax.axis_index("s")
    wid = cid * 16 + sid

    pltpu.sync_copy(src_hbm.at[wid], srcv)
    pltpu.sync_copy(dst_hbm.at[wid], dstv)
    pltpu.sync_copy(tab_hbm, keptt)

    @pl.loop(0, CHUNK, step=16)
    def _(i):
        vals0[pl.ds(i, 16)] = jnp.zeros((16,), jnp.float32)

    @pl.loop(0, ROWS_PER_TILE // CHUNK)
    def _(i):
        pltpu.sync_copy(vals0, acc.at[pl.ds(sid * ROWS_PER_TILE + i * CHUNK, CHUNK)])

    plsc.subcore_barrier()

    @pl.loop(0, CHUNKS_PER_TILE, step=2)
    def _(k):
        @pl.loop(0, CHUNK, step=16)
        def _(j):
            vals0[pl.ds(j, 16)] = plsc.load_gather(keptt, [srcv[k, pl.ds(j, 16)]])

        @pl.when(k > 0)
        def _():
            pltpu.make_async_copy(vals1, acc.at[dstv.at[k - 1]], ss1).wait()

        pltpu.async_copy(vals0, acc.at[dstv.at[k]], ss0, add=True)

        @pl.loop(0, CHUNK, step=16)
        def _(j):
            vals1[pl.ds(j, 16)] = plsc.load_gather(keptt,
                                                   [srcv[k + 1, pl.ds(j, 16)]])

        pltpu.make_async_copy(vals0, acc.at[dstv.at[k]], ss0).wait()
        pltpu.async_copy(vals1, acc.at[dstv.at[k + 1]], ss1, add=True)

    pltpu.make_async_copy(vals1, acc.at[dstv.at[CHUNKS_PER_TILE - 1]], ss1).wait()
    plsc.subcore_barrier()
    pltpu.sync_copy(acc.at[pl.ds(sid * ROWS_PER_TILE, ROWS_PER_TILE)],
                    out_hbm.at[cid, pl.ds(sid * ROWS_PER_TILE, ROWS_PER_TILE)])


def _sweep_scalar(srcp, dstp, tab):
    return pl.kernel(
        _sweep_scalar_body,
        out_type=jax.ShapeDtypeStruct((2, NPAD), jnp.float32),
        mesh=_get_mesh(),
        scratch_types=[
            pltpu.VMEM_SHARED((NPAD,), jnp.float32),
            pltpu.VMEM((NPAD,), jnp.float32),
            pltpu.VMEM((CHUNKS_PER_TILE, CHUNK), jnp.int32),
            pltpu.VMEM((CHUNKS_PER_TILE, CHUNK), jnp.int32),
            pltpu.VMEM((CHUNK,), jnp.float32),
            pltpu.VMEM((CHUNK,), jnp.float32),
            pltpu.SemaphoreType.DMA,
            pltpu.SemaphoreType.DMA,
        ],
        compiler_params=pltpu.CompilerParams(needs_layout_passes=False),
    )(srcp, dstp, tab)


def _mm_body(x_ref, w_ref, o_ref):
    o_ref[...] = jnp.dot(x_ref[...], w_ref[...],
                         preferred_element_type=jnp.float32)


def _tc_mm(x, w):
    return pl.pallas_call(
        _mm_body,
        out_shape=jax.ShapeDtypeStruct((x.shape[0], w.shape[1]), jnp.float32),
    )(x, w)


def _scale_body(xw_ref, kept_ref, degs_ref, y_ref, self_ref, g_ref):
    xw = xw_ref[...]
    kept = kept_ref[...]
    deg = 1.0 + kept * (degs_ref[0] + degs_ref[1])
    g = kept * (1.0 / jnp.sqrt(deg))
    y_ref[...] = g * xw
    self_ref[...] = xw / deg
    g_ref[...] = g


def _tc_scale(xw, kept, degs):
    return pl.pallas_call(
        _scale_body,
        out_shape=[
            jax.ShapeDtypeStruct((NPAD, HP), jnp.float32),
            jax.ShapeDtypeStruct((NPAD, HP), jnp.float32),
            jax.ShapeDtypeStruct((NPAD, 1), jnp.float32),
        ],
    )(xw, kept, degs)


def _post_body(s_ref, self_ref, g_ref, b_ref, kept_ref, xn_ref, y2_ref):
    xn = jnp.maximum(
        g_ref[...] * (s_ref[0] + s_ref[1]) + self_ref[...] + b_ref[...], 0.0)
    xn_ref[...] = xn
    y2_ref[...] = xn * kept_ref[...]


def _tc_post(s, selfterm, g, b, kept):
    return pl.pallas_call(
        _post_body,
        out_shape=[
            jax.ShapeDtypeStruct((NPAD, HP), jnp.float32),
            jax.ShapeDtypeStruct((NPAD, HP), jnp.float32),
        ],
    )(s, selfterm, g, b, kept)


def _score_body(xn_ref, t_ref, kept_ref, wr_ref, wn_ref, bp_ref, score_ref):
    neigh = kept_ref[...] * (t_ref[0] + t_ref[1])
    s = (jnp.dot(xn_ref[...], wr_ref[...], preferred_element_type=jnp.float32)
         + jnp.dot(neigh, wn_ref[...], preferred_element_type=jnp.float32)
         ) + bp_ref[...]
    score_ref[...] = s


def _tc_score(xn, t, kept, wr, wn, bp):
    return pl.pallas_call(
        _score_body,
        out_shape=jax.ShapeDtypeStruct((NPAD, 1), jnp.float32),
    )(xn, t, kept, wr, wn, bp.reshape(1, 1))


def _sortable(f):
    bits = lax.bitcast_convert_type(f, jnp.int32)
    return bits ^ (lax.shift_right_arithmetic(bits, 31) & jnp.int32(0x7FFFFFFF))


def _topk_body(k, sg_ref, kg_ref, out_ref):
    imin = jnp.int32(-2147483648)
    key = jnp.where(kg_ref[...] > 0, _sortable(sg_ref[...]), imin)

    def sbody(bi, t_biased):
        cand = t_biased | lax.shift_left(jnp.int32(1), 31 - bi)
        t_signed = cand ^ imin
        cnt = jnp.sum((key >= t_signed).astype(jnp.int32))
        return jnp.where(cnt >= k, cand, t_biased)

    tb = lax.fori_loop(0, 32, sbody, jnp.int32(0))
    t = tb ^ imin
    strict = jnp.sum((key > t).astype(jnp.int32))
    r = k - strict
    tie = key == t
    idx = (lax.broadcasted_iota(jnp.int32, (80, 128), 0) * 128
           + lax.broadcasted_iota(jnp.int32, (80, 128), 1))
    u = jnp.where(tie, jnp.int32(16383) - idx, jnp.int32(-1))

    def ubody(bi, tu):
        cand = tu | lax.shift_left(jnp.int32(1), 13 - bi)
        cnt = jnp.sum((u >= cand).astype(jnp.int32))
        return jnp.where(cnt >= r, cand, tu)

    tu = lax.fori_loop(0, 14, ubody, jnp.int32(0))
    tau = jnp.where(r > 0, jnp.int32(16383) - tu, jnp.int32(-1))
    out_ref[0, 0] = t
    out_ref[0, 1] = tau
    out_ref[0, 2] = r


def _tc_topk(sg, kg, k):
    return pl.pallas_call(
        functools.partial(_topk_body, k),
        out_shape=jax.ShapeDtypeStruct((1, 4), jnp.int32),
        out_specs=pl.BlockSpec(memory_space=pltpu.SMEM),
    )(sg, kg)


def _gate_body(k, xn_ref, score_ref, kept_ref, scal_ref, xf_ref, kn_ref, ro_ref):
    score = score_ref[...]
    key = jnp.where(kept_ref[...] > 0, _sortable(score), jnp.int32(-2147483648))
    t = scal_ref[0, 0]
    tau = scal_ref[0, 1]
    r = scal_ref[0, 2]
    idx = lax.broadcasted_iota(jnp.int32, (NPAD, 1), 0)
    keptn = ((key > t) | ((key == t) & (idx <= tau) & (r > 0))).astype(
        jnp.float32)
    xf = xn_ref[...] * jnp.tanh(score) * keptn
    xf_ref[...] = xf
    kn_ref[...] = keptn
    mean = jnp.sum(xf, axis=0, keepdims=True) * (1.0 / k)
    mx = jnp.max(jnp.where(keptn > 0, xf, -jnp.inf), axis=0, keepdims=True)
    ro_ref[...] = jnp.concatenate([mean[:, :H], mx[:, :H]], axis=1)


def _tc_gate(xn, score, kept, scal, k):
    return pl.pallas_call(
        functools.partial(_gate_body, k),
        out_shape=[
            jax.ShapeDtypeStruct((NPAD, HP), jnp.float32),
            jax.ShapeDtypeStruct((NPAD, 1), jnp.float32),
            jax.ShapeDtypeStruct((1, 2 * H), jnp.float32),
        ],
        in_specs=[
            pl.BlockSpec(memory_space=pltpu.VMEM),
            pl.BlockSpec(memory_space=pltpu.VMEM),
            pl.BlockSpec(memory_space=pltpu.VMEM),
            pl.BlockSpec(memory_space=pltpu.SMEM),
        ],
    )(xn, score, kept, scal)


def _pad_cols(w, rows=None):
    out_rows = w.shape[0] if rows is None else rows
    z = jnp.zeros((out_rows, HP), jnp.float32)
    return z.at[:w.shape[0], :w.shape[1]].set(w)


def kernel(x, edge_index, batch, W1, b1, W2, b2, W3, b3, W4, b4,
           Wr1, Wn1, bp1, Wr2, Wn2, bp2, Wr3, Wn3, bp3, Wr4, Wn4, bp4):
    src = edge_index[0]
    dst = edge_index[1]
    npad_e = EP - E
    pad_iota = jnp.arange(npad_e, dtype=jnp.int32) % 128
    srcf = jnp.concatenate([src, pad_iota])
    srcp = srcf.reshape(NTILES, CHUNKS_PER_TILE, CHUNK)
    dstp = jnp.concatenate([dst, N + pad_iota]).reshape(
        NTILES, CHUNKS_PER_TILE, CHUNK)

    xp = jnp.zeros((NPAD, HP), jnp.float32).at[:N, :x.shape[1]].set(x)
    kept = jnp.zeros((NPAD, 1), jnp.float32).at[:N].set(1.0)

    Ws = ((_pad_cols(W1, 128), _pad_cols(b1.reshape(1, H))),
          (_pad_cols(W2, 128), _pad_cols(b2.reshape(1, H))),
          (_pad_cols(W3, 128), _pad_cols(b3.reshape(1, H))),
          (_pad_cols(W4, 128), _pad_cols(b4.reshape(1, H))))
    def _pad_rows(w):
        return jnp.zeros((HP, 1), jnp.float32).at[:w.shape[0]].set(w)

    Ps = ((_pad_rows(Wr1), _pad_rows(Wn1), bp1),
          (_pad_rows(Wr2), _pad_rows(Wn2), bp2),
          (_pad_rows(Wr3), _pad_rows(Wn3), bp3),
          (_pad_rows(Wr4), _pad_rows(Wn4), bp4))

    out = jnp.zeros((1, 2 * H), jnp.float32)
    xc = xp
    for i in range(4):
        W, b = Ws[i]
        Wr, Wn, bp = Ps[i]
        k = K_SIZES[i]
        xw = _tc_mm(xc, W)
        degs = _sweep_scalar(srcp, dstp, kept.reshape(NPAD))
        y, selfterm, g = _tc_scale(xw, kept, degs.reshape(2, NPAD, 1))
        s = _sweep_rows(srcf, dstp, y)
        xn, y2 = _tc_post(s, selfterm, g, b, kept)
        t = _sweep_rows(srcf, dstp, y2)
        score = _tc_score(xn, t, kept, Wr, Wn, bp)
        scal = _tc_topk(score.reshape(80, 128), kept.reshape(80, 128), k)
        xc, kept, ro = _tc_gate(xn, score, kept, scal, k)
        out = out + ro
    return out


# speedup vs baseline: 31.1980x; 1.0092x over previous
import functools
import math

import jax
import jax.numpy as jnp
from jax import lax
from jax.experimental import pallas as pl
from jax.experimental.pallas import tpu as pltpu
from jax.experimental.pallas import tpu_sc as plsc

N = 10000
E = 320000
H = 64
HP = 128
NPAD = 10240
NTILES = 32
CHUNK = 128
CHUNKS_PER_TILE = 80
EPT = CHUNK * CHUNKS_PER_TILE
EP = EPT * NTILES
ROWS_PER_TILE = NPAD // 16
K_SIZES = (5000, 2500, 1250, 625)


@functools.cache
def _get_mesh():
    return plsc.VectorSubcoreMesh(core_axis_name="c", subcore_axis_name="s",
                                  num_cores=2, num_subcores=16)


def _sweep_rows_body(src_hbm, dst_hbm, tab_hbm, out_hbm, acc,
                     dstv, sa, sb, rows0, rows1,
                     sg0, sg1, ss0, ss1, sia, sib):
    cid = lax.axis_index("c")
    sid = lax.axis_index("s")
    wid = cid * 16 + sid
    CPT = CHUNKS_PER_TILE

    pltpu.sync_copy(dst_hbm.at[wid], dstv)

    @pl.loop(0, CHUNK)
    def _(i):
        for j in range(HP // 16):
            rows0[i, pl.ds(j * 16, 16)] = jnp.zeros((16,), jnp.float32)

    @pl.loop(0, ROWS_PER_TILE // CHUNK)
    def _(i):
        pltpu.sync_copy(rows0, acc.at[pl.ds(sid * ROWS_PER_TILE + i * CHUNK, CHUNK)])

    plsc.subcore_barrier()

    def idx_load(k, buf, sem):
        pltpu.async_copy(src_hbm.at[pl.ds(wid * EPT + k * CHUNK, CHUNK)],
                         buf, sem)

    def idx_wait(k, buf, sem):
        pltpu.make_async_copy(src_hbm.at[pl.ds(wid * EPT + k * CHUNK, CHUNK)],
                              buf, sem).wait()

    pltpu.sync_copy(src_hbm.at[pl.ds(wid * EPT, CHUNK)], sa)
    pltpu.async_copy(tab_hbm.at[sa], rows0, sg0)
    idx_load(1, sb, sib)

    @pl.loop(0, CPT, step=2)
    def _(k):
        pltpu.make_async_copy(tab_hbm.at[sa], rows0, sg0).wait()
        idx_wait(k + 1, sb, sib)

        @pl.when(k > 0)
        def _():
            pltpu.make_async_copy(rows1, acc.at[dstv.at[k - 1]], ss1).wait()

        pltpu.async_copy(tab_hbm.at[sb], rows1, sg1)
        pltpu.async_copy(rows0, acc.at[dstv.at[k]], ss0, add=True)

        @pl.when(k + 2 < CPT)
        def _():
            idx_load(k + 2, sa, sia)

        pltpu.make_async_copy(tab_hbm.at[sb], rows1, sg1).wait()
        pltpu.make_async_copy(rows0, acc.at[dstv.at[k]], ss0).wait()

        @pl.when(k + 2 < CPT)
        def _():
            idx_wait(k + 2, sa, sia)
            pltpu.async_copy(tab_hbm.at[sa], rows0, sg0)

        pltpu.async_copy(rows1, acc.at[dstv.at[k + 1]], ss1, add=True)

        @pl.when(k + 3 < CPT)
        def _():
            idx_load(k + 3, sb, sib)

    pltpu.make_async_copy(rows1, acc.at[dstv.at[CPT - 1]], ss1).wait()
    plsc.subcore_barrier()
    pltpu.sync_copy(acc.at[pl.ds(sid * ROWS_PER_TILE, ROWS_PER_TILE)],
                    out_hbm.at[cid, pl.ds(sid * ROWS_PER_TILE, ROWS_PER_TILE)])


def _sweep_rows(srcp, dstp, tab):
    return pl.kernel(
        _sweep_rows_body,
        out_type=jax.ShapeDtypeStruct((2, NPAD, HP), jnp.float32),
        mesh=_get_mesh(),
        scratch_types=[
            pltpu.VMEM_SHARED((NPAD, HP), jnp.float32),
            pltpu.VMEM((CHUNKS_PER_TILE, CHUNK), jnp.int32),
            pltpu.VMEM((CHUNK,), jnp.int32),
            pltpu.VMEM((CHUNK,), jnp.int32),
            pltpu.VMEM((CHUNK, HP), jnp.float32),
            pltpu.VMEM((CHUNK, HP), jnp.float32),
            pltpu.SemaphoreType.DMA,
            pltpu.SemaphoreType.DMA,
            pltpu.SemaphoreType.DMA,
            pltpu.SemaphoreType.DMA,
            pltpu.SemaphoreType.DMA,
            pltpu.SemaphoreType.DMA,
        ],
    )(srcp, dstp, tab)


_RING = 8


def _sweep_scalar_body(src_hbm, dst_hbm, tab_hbm, out_hbm,
                       acc, keptt, srcv, dstv, vals, *sems):
    cid = lax.axis_index("c")
    sid = lax.axis_index("s")
    wid = cid * 16 + sid

    pltpu.sync_copy(src_hbm.at[wid], srcv)
    pltpu.sync_copy(dst_hbm.at[wid], dstv)
    pltpu.sync_copy(tab_hbm, keptt)

    @pl.loop(0, CHUNK, step=16)
    def _(i):
        vals[0, pl.ds(i, 16)] = jnp.zeros((16,), jnp.float32)

    @pl.loop(0, ROWS_PER_TILE // CHUNK)
    def _(i):
        pltpu.sync_copy(vals.at[0],
                        acc.at[pl.ds(sid * ROWS_PER_TILE + i * CHUNK, CHUNK)])

    plsc.subcore_barrier()

    @pl.loop(0, CHUNKS_PER_TILE, step=_RING)
    def _(k):
        for j in range(_RING):
            @pl.when(k > 0)
            def _():
                pltpu.make_async_copy(
                    vals.at[j], acc.at[dstv.at[k - _RING + j]], sems[j]).wait()

            @pl.loop(0, CHUNK, step=16)
            def _(i):
                vals[j, pl.ds(i, 16)] = plsc.load_gather(
                    keptt, [srcv[k + j, pl.ds(i, 16)]])

            pltpu.async_copy(vals.at[j], acc.at[dstv.at[k + j]], sems[j],
                             add=True)

    for j in range(_RING):
        pltpu.make_async_copy(
            vals.at[j], acc.at[dstv.at[CHUNKS_PER_TILE - _RING + j]],
            sems[j]).wait()

    plsc.subcore_barrier()
    pltpu.sync_copy(acc.at[pl.ds(sid * ROWS_PER_TILE, ROWS_PER_TILE)],
                    out_hbm.at[cid, pl.ds(sid * ROWS_PER_TILE, ROWS_PER_TILE)])


def _sweep_scalar(srcp, dstp, tab):
    return pl.kernel(
        _sweep_scalar_body,
        out_type=jax.ShapeDtypeStruct((2, NPAD), jnp.float32),
        mesh=_get_mesh(),
        scratch_types=[
            pltpu.VMEM_SHARED((NPAD,), jnp.float32),
            pltpu.VMEM((NPAD,), jnp.float32),
            pltpu.VMEM((CHUNKS_PER_TILE, CHUNK), jnp.int32),
            pltpu.VMEM((CHUNKS_PER_TILE, CHUNK), jnp.int32),
            pltpu.VMEM((_RING, CHUNK), jnp.float32),
        ] + [pltpu.SemaphoreType.DMA] * _RING,
        compiler_params=pltpu.CompilerParams(needs_layout_passes=False),
    )(srcp, dstp, tab)


def _mm_body(x_ref, w_ref, o_ref):
    o_ref[...] = jnp.dot(x_ref[...], w_ref[...],
                         preferred_element_type=jnp.float32)


def _tc_mm(x, w):
    return pl.pallas_call(
        _mm_body,
        out_shape=jax.ShapeDtypeStruct((x.shape[0], w.shape[1]), jnp.float32),
    )(x, w)


def _scale_body(xw_ref, kept_ref, degs_ref, y_ref, self_ref, g_ref):
    xw = xw_ref[...]
    kept = kept_ref[...]
    deg = 1.0 + kept * (degs_ref[0] + degs_ref[1])
    g = kept * (1.0 / jnp.sqrt(deg))
    y_ref[...] = g * xw
    self_ref[...] = xw / deg
    g_ref[...] = g


def _tc_scale(xw, kept, degs):
    return pl.pallas_call(
        _scale_body,
        out_shape=[
            jax.ShapeDtypeStruct((NPAD, HP), jnp.float32),
            jax.ShapeDtypeStruct((NPAD, HP), jnp.float32),
            jax.ShapeDtypeStruct((NPAD, 1), jnp.float32),
        ],
    )(xw, kept, degs)


def _post_body(s_ref, self_ref, g_ref, b_ref, kept_ref, xn_ref, y2_ref):
    xn = jnp.maximum(
        g_ref[...] * (s_ref[0] + s_ref[1]) + self_ref[...] + b_ref[...], 0.0)
    xn_ref[...] = xn
    y2_ref[...] = xn * kept_ref[...]


def _tc_post(s, selfterm, g, b, kept):
    return pl.pallas_call(
        _post_body,
        out_shape=[
            jax.ShapeDtypeStruct((NPAD, HP), jnp.float32),
            jax.ShapeDtypeStruct((NPAD, HP), jnp.float32),
        ],
    )(s, selfterm, g, b, kept)


def _score_body(xn_ref, t_ref, kept_ref, wr_ref, wn_ref, bp_ref, score_ref):
    neigh = kept_ref[...] * (t_ref[0] + t_ref[1])
    s = (jnp.dot(xn_ref[...], wr_ref[...], preferred_element_type=jnp.float32)
         + jnp.dot(neigh, wn_ref[...], preferred_element_type=jnp.float32)
         ) + bp_ref[...]
    score_ref[...] = s


def _tc_score(xn, t, kept, wr, wn, bp):
    return pl.pallas_call(
        _score_body,
        out_shape=jax.ShapeDtypeStruct((NPAD, 1), jnp.float32),
    )(xn, t, kept, wr, wn, bp.reshape(1, 1))


def _sortable(f):
    bits = lax.bitcast_convert_type(f, jnp.int32)
    return bits ^ (lax.shift_right_arithmetic(bits, 31) & jnp.int32(0x7FFFFFFF))


def _topk_body(k, sg_ref, kg_ref, out_ref):
    imin = jnp.int32(-2147483648)
    key = jnp.where(kg_ref[...] > 0, _sortable(sg_ref[...]), imin)

    def sbody(bi, t_biased):
        cand = t_biased | lax.shift_left(jnp.int32(1), 31 - bi)
        t_signed = cand ^ imin
        cnt = jnp.sum((key >= t_signed).astype(jnp.int32))
        return jnp.where(cnt >= k, cand, t_biased)

    tb = lax.fori_loop(0, 32, sbody, jnp.int32(0))
    t = tb ^ imin
    strict = jnp.sum((key > t).astype(jnp.int32))
    r = k - strict
    tie = key == t
    idx = (lax.broadcasted_iota(jnp.int32, (80, 128), 0) * 128
           + lax.broadcasted_iota(jnp.int32, (80, 128), 1))
    u = jnp.where(tie, jnp.int32(16383) - idx, jnp.int32(-1))

    def ubody(bi, tu):
        cand = tu | lax.shift_left(jnp.int32(1), 13 - bi)
        cnt = jnp.sum((u >= cand).astype(jnp.int32))
        return jnp.where(cnt >= r, cand, tu)

    tu = lax.fori_loop(0, 14, ubody, jnp.int32(0))
    tau = jnp.where(r > 0, jnp.int32(16383) - tu, jnp.int32(-1))
    out_ref[0, 0] = t
    out_ref[0, 1] = tau
    out_ref[0, 2] = r


def _tc_topk(sg, kg, k):
    return pl.pallas_call(
        functools.partial(_topk_body, k),
        out_shape=jax.ShapeDtypeStruct((1, 4), jnp.int32),
        out_specs=pl.BlockSpec(memory_space=pltpu.SMEM),
    )(sg, kg)


def _gate_body(k, xn_ref, score_ref, kept_ref, scal_ref, xf_ref, kn_ref, ro_ref):
    score = score_ref[...]
    key = jnp.where(kept_ref[...] > 0, _sortable(score), jnp.int32(-2147483648))
    t = scal_ref[0, 0]
    tau = scal_ref[0, 1]
    r = scal_ref[0, 2]
    idx = lax.broadcasted_iota(jnp.int32, (NPAD, 1), 0)
    keptn = ((key > t) | ((key == t) & (idx <= tau) & (r > 0))).astype(
        jnp.float32)
    xf = xn_ref[...] * jnp.tanh(score) * keptn
    xf_ref[...] = xf
    kn_ref[...] = keptn
    mean = jnp.sum(xf, axis=0, keepdims=True) * (1.0 / k)
    mx = jnp.max(jnp.where(keptn > 0, xf, -jnp.inf), axis=0, keepdims=True)
    ro_ref[...] = jnp.concatenate([mean[:, :H], mx[:, :H]], axis=1)


def _tc_gate(xn, score, kept, scal, k):
    return pl.pallas_call(
        functools.partial(_gate_body, k),
        out_shape=[
            jax.ShapeDtypeStruct((NPAD, HP), jnp.float32),
            jax.ShapeDtypeStruct((NPAD, 1), jnp.float32),
            jax.ShapeDtypeStruct((1, 2 * H), jnp.float32),
        ],
        in_specs=[
            pl.BlockSpec(memory_space=pltpu.VMEM),
            pl.BlockSpec(memory_space=pltpu.VMEM),
            pl.BlockSpec(memory_space=pltpu.VMEM),
            pl.BlockSpec(memory_space=pltpu.SMEM),
        ],
    )(xn, score, kept, scal)


def _pad_cols(w, rows=None):
    out_rows = w.shape[0] if rows is None else rows
    z = jnp.zeros((out_rows, HP), jnp.float32)
    return z.at[:w.shape[0], :w.shape[1]].set(w)


def kernel(x, edge_index, batch, W1, b1, W2, b2, W3, b3, W4, b4,
           Wr1, Wn1, bp1, Wr2, Wn2, bp2, Wr3, Wn3, bp3, Wr4, Wn4, bp4):
    src = edge_index[0]
    dst = edge_index[1]
    npad_e = EP - E
    pad_iota = jnp.arange(npad_e, dtype=jnp.int32) % 128
    srcf = jnp.concatenate([src, pad_iota])
    srcp = srcf.reshape(NTILES, CHUNKS_PER_TILE, CHUNK)
    dstp = jnp.concatenate([dst, N + pad_iota]).reshape(
        NTILES, CHUNKS_PER_TILE, CHUNK)

    xp = jnp.zeros((NPAD, HP), jnp.float32).at[:N, :x.shape[1]].set(x)
    kept = jnp.zeros((NPAD, 1), jnp.float32).at[:N].set(1.0)

    Ws = ((_pad_cols(W1, 128), _pad_cols(b1.reshape(1, H))),
          (_pad_cols(W2, 128), _pad_cols(b2.reshape(1, H))),
          (_pad_cols(W3, 128), _pad_cols(b3.reshape(1, H))),
          (_pad_cols(W4, 128), _pad_cols(b4.reshape(1, H))))
    def _pad_rows(w):
        return jnp.zeros((HP, 1), jnp.float32).at[:w.shape[0]].set(w)

    Ps = ((_pad_rows(Wr1), _pad_rows(Wn1), bp1),
          (_pad_rows(Wr2), _pad_rows(Wn2), bp2),
          (_pad_rows(Wr3), _pad_rows(Wn3), bp3),
          (_pad_rows(Wr4), _pad_rows(Wn4), bp4))

    out = jnp.zeros((1, 2 * H), jnp.float32)
    xc = xp
    for i in range(4):
        W, b = Ws[i]
        Wr, Wn, bp = Ps[i]
        k = K_SIZES[i]
        xw = _tc_mm(xc, W)
        degs = _sweep_scalar(srcp, dstp, kept.reshape(NPAD))
        y, selfterm, g = _tc_scale(xw, kept, degs.reshape(2, NPAD, 1))
        s = _sweep_rows(srcf, dstp, y)
        xn, y2 = _tc_post(s, selfterm, g, b, kept)
        t = _sweep_rows(srcf, dstp, y2)
        score = _tc_score(xn, t, kept, Wr, Wn, bp)
        scal = _tc_topk(score.reshape(80, 128), kept.reshape(80, 128), k)
        xc, kept, ro = _tc_gate(xn, score, kept, scal, k)
        out = out + ro
    return out


# speedup vs baseline: 31.2965x; 1.0032x over previous
import functools
import math

import jax
import jax.numpy as jnp
from jax import lax
from jax.experimental import pallas as pl
from jax.experimental.pallas import tpu as pltpu
from jax.experimental.pallas import tpu_sc as plsc

N = 10000
E = 320000
H = 64
HP = 128
NPAD = 10240
NTILES = 32
CHUNK = 128
CHUNKS_PER_TILE = 80
EPT = CHUNK * CHUNKS_PER_TILE
EP = EPT * NTILES
ROWS_PER_TILE = NPAD // 16
K_SIZES = (5000, 2500, 1250, 625)


@functools.cache
def _get_mesh():
    return plsc.VectorSubcoreMesh(core_axis_name="c", subcore_axis_name="s",
                                  num_cores=2, num_subcores=16)


def _sweep_rows_body(src_hbm, dst_hbm, tab_hbm, out_hbm, acc,
                     dstv, sa, sb, rows0, rows1,
                     sg0, sg1, ss0, ss1, sia, sib):
    cid = lax.axis_index("c")
    sid = lax.axis_index("s")
    wid = cid * 16 + sid
    CPT = CHUNKS_PER_TILE

    pltpu.sync_copy(dst_hbm.at[wid], dstv)

    @pl.loop(0, CHUNK)
    def _(i):
        for j in range(HP // 16):
            rows0[i, pl.ds(j * 16, 16)] = jnp.zeros((16,), jnp.float32)

    @pl.loop(0, ROWS_PER_TILE // CHUNK)
    def _(i):
        pltpu.sync_copy(rows0, acc.at[pl.ds(sid * ROWS_PER_TILE + i * CHUNK, CHUNK)])

    plsc.subcore_barrier()

    def idx_load(k, buf, sem):
        pltpu.async_copy(src_hbm.at[pl.ds(wid * EPT + k * CHUNK, CHUNK)],
                         buf, sem)

    def idx_wait(k, buf, sem):
        pltpu.make_async_copy(src_hbm.at[pl.ds(wid * EPT + k * CHUNK, CHUNK)],
                              buf, sem).wait()

    pltpu.sync_copy(src_hbm.at[pl.ds(wid * EPT, CHUNK)], sa)
    pltpu.async_copy(tab_hbm.at[sa], rows0, sg0)
    idx_load(1, sb, sib)

    @pl.loop(0, CPT, step=2)
    def _(k):
        pltpu.make_async_copy(tab_hbm.at[sa], rows0, sg0).wait()
        idx_wait(k + 1, sb, sib)

        @pl.when(k > 0)
        def _():
            pltpu.make_async_copy(rows1, acc.at[dstv.at[k - 1]], ss1).wait()

        pltpu.async_copy(tab_hbm.at[sb], rows1, sg1)
        pltpu.async_copy(rows0, acc.at[dstv.at[k]], ss0, add=True)

        @pl.when(k + 2 < CPT)
        def _():
            idx_load(k + 2, sa, sia)

        pltpu.make_async_copy(tab_hbm.at[sb], rows1, sg1).wait()
        pltpu.make_async_copy(rows0, acc.at[dstv.at[k]], ss0).wait()

        @pl.when(k + 2 < CPT)
        def _():
            idx_wait(k + 2, sa, sia)
            pltpu.async_copy(tab_hbm.at[sa], rows0, sg0)

        pltpu.async_copy(rows1, acc.at[dstv.at[k + 1]], ss1, add=True)

        @pl.when(k + 3 < CPT)
        def _():
            idx_load(k + 3, sb, sib)

    pltpu.make_async_copy(rows1, acc.at[dstv.at[CPT - 1]], ss1).wait()
    plsc.subcore_barrier()
    pltpu.sync_copy(acc.at[pl.ds(sid * ROWS_PER_TILE, ROWS_PER_TILE)],
                    out_hbm.at[cid, pl.ds(sid * ROWS_PER_TILE, ROWS_PER_TILE)])


def _sweep_rows(srcp, dstp, tab):
    return pl.kernel(
        _sweep_rows_body,
        out_type=jax.ShapeDtypeStruct((2, NPAD, HP), jnp.float32),
        mesh=_get_mesh(),
        scratch_types=[
            pltpu.VMEM_SHARED((NPAD, HP), jnp.float32),
            pltpu.VMEM((CHUNKS_PER_TILE, CHUNK), jnp.int32),
            pltpu.VMEM((CHUNK,), jnp.int32),
            pltpu.VMEM((CHUNK,), jnp.int32),
            pltpu.VMEM((CHUNK, HP), jnp.float32),
            pltpu.VMEM((CHUNK, HP), jnp.float32),
            pltpu.SemaphoreType.DMA,
            pltpu.SemaphoreType.DMA,
            pltpu.SemaphoreType.DMA,
            pltpu.SemaphoreType.DMA,
            pltpu.SemaphoreType.DMA,
            pltpu.SemaphoreType.DMA,
        ],
    )(srcp, dstp, tab)


_RING = 8


def _sweep_scalar_body(src_hbm, dst_hbm, tab_hbm, out_hbm,
                       acc, keptt, srcv, dstv, vals, *sems):
    cid = lax.axis_index("c")
    sid = lax.axis_index("s")
    wid = cid * 16 + sid

    pltpu.sync_copy(src_hbm.at[wid], srcv)
    pltpu.sync_copy(dst_hbm.at[wid], dstv)
    pltpu.sync_copy(tab_hbm, keptt)

    @pl.loop(0, CHUNK, step=16)
    def _(i):
        vals[0, pl.ds(i, 16)] = jnp.zeros((16,), jnp.float32)

    @pl.loop(0, ROWS_PER_TILE // CHUNK)
    def _(i):
        pltpu.sync_copy(vals.at[0],
                        acc.at[pl.ds(sid * ROWS_PER_TILE + i * CHUNK, CHUNK)])

    plsc.subcore_barrier()

    @pl.loop(0, CHUNKS_PER_TILE, step=_RING)
    def _(k):
        for j in range(_RING):
            @pl.when(k > 0)
            def _():
                pltpu.make_async_copy(
                    vals.at[j], acc.at[dstv.at[k - _RING + j]], sems[j]).wait()

            @pl.loop(0, CHUNK, step=16)
            def _(i):
                vals[j, pl.ds(i, 16)] = plsc.load_gather(
                    keptt, [srcv[k + j, pl.ds(i, 16)]])

            pltpu.async_copy(vals.at[j], acc.at[dstv.at[k + j]], sems[j],
                             add=True)

    for j in range(_RING):
        pltpu.make_async_copy(
            vals.at[j], acc.at[dstv.at[CHUNKS_PER_TILE - _RING + j]],
            sems[j]).wait()

    plsc.subcore_barrier()
    pltpu.sync_copy(acc.at[pl.ds(sid * ROWS_PER_TILE, ROWS_PER_TILE)],
                    out_hbm.at[cid, pl.ds(sid * ROWS_PER_TILE, ROWS_PER_TILE)])


def _sweep_scalar(srcp, dstp, tab):
    return pl.kernel(
        _sweep_scalar_body,
        out_type=jax.ShapeDtypeStruct((2, NPAD), jnp.float32),
        mesh=_get_mesh(),
        scratch_types=[
            pltpu.VMEM_SHARED((NPAD,), jnp.float32),
            pltpu.VMEM((NPAD,), jnp.float32),
            pltpu.VMEM((CHUNKS_PER_TILE, CHUNK), jnp.int32),
            pltpu.VMEM((CHUNKS_PER_TILE, CHUNK), jnp.int32),
            pltpu.VMEM((_RING, CHUNK), jnp.float32),
        ] + [pltpu.SemaphoreType.DMA] * _RING,
        compiler_params=pltpu.CompilerParams(needs_layout_passes=False),
    )(srcp, dstp, tab)


def _mm_body(x_ref, w_ref, o_ref):
    o_ref[...] = jnp.dot(x_ref[...], w_ref[...],
                         preferred_element_type=jnp.float32)


def _tc_mm(x, w):
    return pl.pallas_call(
        _mm_body,
        out_shape=jax.ShapeDtypeStruct((x.shape[0], w.shape[1]), jnp.float32),
    )(x, w)


def _scale_body(xw_ref, kept_ref, degs_ref, y_ref, self_ref, g_ref):
    xw = xw_ref[...]
    kept = kept_ref[...]
    deg = 1.0 + kept * (degs_ref[0] + degs_ref[1])
    g = kept * (1.0 / jnp.sqrt(deg))
    y_ref[...] = g * xw
    self_ref[...] = xw / deg
    g_ref[...] = g


def _tc_scale(xw, kept, degs):
    return pl.pallas_call(
        _scale_body,
        out_shape=[
            jax.ShapeDtypeStruct((NPAD, HP), jnp.float32),
            jax.ShapeDtypeStruct((NPAD, HP), jnp.float32),
            jax.ShapeDtypeStruct((NPAD, 1), jnp.float32),
        ],
    )(xw, kept, degs)


def _post_body(s_ref, self_ref, g_ref, b_ref, kept_ref, xn_ref, y2_ref):
    xn = jnp.maximum(
        g_ref[...] * (s_ref[0] + s_ref[1]) + self_ref[...] + b_ref[...], 0.0)
    xn_ref[...] = xn
    y2_ref[...] = xn * kept_ref[...]


def _tc_post(s, selfterm, g, b, kept):
    return pl.pallas_call(
        _post_body,
        out_shape=[
            jax.ShapeDtypeStruct((NPAD, HP), jnp.float32),
            jax.ShapeDtypeStruct((NPAD, HP), jnp.float32),
        ],
    )(s, selfterm, g, b, kept)


def _score_body(xn_ref, t_ref, kept_ref, wr_ref, wn_ref, bp_ref, score_ref):
    neigh = kept_ref[...] * (t_ref[0] + t_ref[1])
    s = (jnp.dot(xn_ref[...], wr_ref[...], preferred_element_type=jnp.float32)
         + jnp.dot(neigh, wn_ref[...], preferred_element_type=jnp.float32)
         ) + bp_ref[...]
    score_ref[...] = s


def _tc_score(xn, t, kept, wr, wn, bp):
    return pl.pallas_call(
        _score_body,
        out_shape=jax.ShapeDtypeStruct((NPAD, 1), jnp.float32),
    )(xn, t, kept, wr, wn, bp.reshape(1, 1))


def _sortable(f):
    bits = lax.bitcast_convert_type(f, jnp.int32)
    return bits ^ (lax.shift_right_arithmetic(bits, 31) & jnp.int32(0x7FFFFFFF))


def _gate_body(k, xn_ref, score_ref, kept_ref, sg_ref, kg_ref, w_ref,
               xw_ref, kn_ref, ro_ref):
    imin = jnp.int32(-2147483648)
    key8 = jnp.where(kg_ref[...] > 0, _sortable(sg_ref[...]), imin)

    def sbody(bi, t_biased):
        cand = t_biased | lax.shift_left(jnp.int32(1), 31 - bi)
        t_signed = cand ^ imin
        cnt = jnp.sum((key8 >= t_signed).astype(jnp.int32))
        return jnp.where(cnt >= k, cand, t_biased)

    tb = lax.fori_loop(0, 32, sbody, jnp.int32(0))
    t = tb ^ imin
    strict = jnp.sum((key8 > t).astype(jnp.int32))
    r = k - strict
    tie8 = key8 == t
    idx8 = (lax.broadcasted_iota(jnp.int32, (80, 128), 0) * 128
            + lax.broadcasted_iota(jnp.int32, (80, 128), 1))
    u = jnp.where(tie8, jnp.int32(16383) - idx8, jnp.int32(-1))

    def ubody(bi, tu):
        cand = tu | lax.shift_left(jnp.int32(1), 13 - bi)
        cnt = jnp.sum((u >= cand).astype(jnp.int32))
        return jnp.where(cnt >= r, cand, tu)

    tu = lax.fori_loop(0, 14, ubody, jnp.int32(0))
    tau = jnp.where(r > 0, jnp.int32(16383) - tu, jnp.int32(-1))

    score = score_ref[...]
    key = jnp.where(kept_ref[...] > 0, _sortable(score), imin)
    idx = lax.broadcasted_iota(jnp.int32, (NPAD, 1), 0)
    keptn = ((key > t) | ((key == t) & (idx <= tau) & (r > 0))).astype(
        jnp.float32)
    xf = xn_ref[...] * jnp.tanh(score) * keptn
    kn_ref[...] = keptn
    mean = jnp.sum(xf, axis=0, keepdims=True) * (1.0 / k)
    mx = jnp.max(jnp.where(keptn > 0, xf, -jnp.inf), axis=0, keepdims=True)
    ro_ref[...] = jnp.concatenate([mean[:, :H], mx[:, :H]], axis=1)
    xw_ref[...] = jnp.dot(xf, w_ref[...], preferred_element_type=jnp.float32)


def _tc_gate(xn, score, kept, sg, kg, wnext, k):
    return pl.pallas_call(
        functools.partial(_gate_body, k),
        out_shape=[
            jax.ShapeDtypeStruct((NPAD, HP), jnp.float32),
            jax.ShapeDtypeStruct((NPAD, 1), jnp.float32),
            jax.ShapeDtypeStruct((1, 2 * H), jnp.float32),
        ],
    )(xn, score, kept, sg, kg, wnext)


def _pad_cols(w, rows=None):
    out_rows = w.shape[0] if rows is None else rows
    z = jnp.zeros((out_rows, HP), jnp.float32)
    return z.at[:w.shape[0], :w.shape[1]].set(w)


def kernel(x, edge_index, batch, W1, b1, W2, b2, W3, b3, W4, b4,
           Wr1, Wn1, bp1, Wr2, Wn2, bp2, Wr3, Wn3, bp3, Wr4, Wn4, bp4):
    src = edge_index[0]
    dst = edge_index[1]
    npad_e = EP - E
    pad_iota = jnp.arange(npad_e, dtype=jnp.int32) % 128
    srcf = jnp.concatenate([src, pad_iota])
    srcp = srcf.reshape(NTILES, CHUNKS_PER_TILE, CHUNK)
    dstp = jnp.concatenate([dst, N + pad_iota]).reshape(
        NTILES, CHUNKS_PER_TILE, CHUNK)

    xp = jnp.zeros((NPAD, HP), jnp.float32).at[:N, :x.shape[1]].set(x)
    kept = jnp.zeros((NPAD, 1), jnp.float32).at[:N].set(1.0)

    Ws = ((_pad_cols(W1, 128), _pad_cols(b1.reshape(1, H))),
          (_pad_cols(W2, 128), _pad_cols(b2.reshape(1, H))),
          (_pad_cols(W3, 128), _pad_cols(b3.reshape(1, H))),
          (_pad_cols(W4, 128), _pad_cols(b4.reshape(1, H))))
    def _pad_rows(w):
        return jnp.zeros((HP, 1), jnp.float32).at[:w.shape[0]].set(w)

    Ps = ((_pad_rows(Wr1), _pad_rows(Wn1), bp1),
          (_pad_rows(Wr2), _pad_rows(Wn2), bp2),
          (_pad_rows(Wr3), _pad_rows(Wn3), bp3),
          (_pad_rows(Wr4), _pad_rows(Wn4), bp4))

    out = jnp.zeros((1, 2 * H), jnp.float32)
    xw = _tc_mm(xp, Ws[0][0])
    for i in range(4):
        b = Ws[i][1]
        Wr, Wn, bp = Ps[i]
        k = K_SIZES[i]
        wnext = Ws[i + 1][0] if i < 3 else jnp.zeros((HP, HP), jnp.float32)
        degs = _sweep_scalar(srcp, dstp, kept.reshape(NPAD))
        y, selfterm, g = _tc_scale(xw, kept, degs.reshape(2, NPAD, 1))
        s = _sweep_rows(srcf, dstp, y)
        xn, y2 = _tc_post(s, selfterm, g, b, kept)
        t = _sweep_rows(srcf, dstp, y2)
        score = _tc_score(xn, t, kept, Wr, Wn, bp)
        xw, kept, ro = _tc_gate(xn, score, kept, score.reshape(80, 128),
                                kept.reshape(80, 128), wnext, k)
        out = out + ro
    return out
